# baseline (speedup 1.0000x reference)
import functools

import jax
import jax.numpy as jnp
from jax import lax
from jax.experimental import pallas as pl
from jax.experimental.pallas import tpu as pltpu
from jax.experimental.pallas import tpu_sc as plsc

N = 10000
E = 320000
F = 128
H = 128
R = 4
G = 64

NC = 2
NS = 16
L = 16

EPT = E // NS
C = 80
NCH = EPT // C
ROWS_PT = 632
N_PAD = ROWS_PT * NS
DUMMY = N

_f32 = jnp.float32
_i32 = jnp.int32


def _make_agg_body(with_cnt):
    def body(*refs):
        es_hbm, ed_hbm, et_hbm, y_hbm, a_hbm = refs[:5]
        cnt_hbm = refs[5] if with_cnt else None
        (srcv, typv, dstv, giv, siv, buf, onesv, zbuf, zcnt, cbuf,
         sem_i, sem_g, sem_s, acc, cacc) = refs[6 if with_cnt else 5:]
        core = lax.axis_index("c")
        tile = lax.axis_index("s")
        row0 = tile * ROWS_PT
        ebase0 = tile * EPT

        @pl.loop(0, zbuf.shape[0])
        def _(r):
            for j in range(F // L):
                zbuf[r, pl.ds(j * L, L)] = jnp.zeros((L,), _f32)

        if with_cnt:
            @pl.loop(0, zcnt.shape[0], step=L)
            def _(i):
                zcnt[pl.ds(i, L)] = jnp.zeros((L,), _f32)

            @pl.loop(0, C, step=L)
            def _(i):
                onesv[pl.ds(i, L)] = jnp.ones((L,), _f32)

        def load_idx(b, base):
            return (pltpu.async_copy(es_hbm.at[pl.ds(base, C)], srcv[b], sem_i[b]),
                    pltpu.async_copy(ed_hbm.at[pl.ds(base, C)], dstv[b], sem_i[b]),
                    pltpu.async_copy(et_hbm.at[pl.ds(base, C)], typv[b], sem_i[b]))

        def fixup(b, rel):
            @pl.loop(0, C, step=L)
            def _(i):
                t = typv[b][pl.ds(i, L)]
                s = srcv[b][pl.ds(i, L)]
                d = dstv[b][pl.ds(i, L)]
                giv[b][pl.ds(i, L)] = t * N + s
                own = t == rel
                siv[b][pl.ds(i, L)] = jnp.where(own, d, DUMMY)

        for p in range(2):
            rel = core * 2 + p

            zc = [(i * 64, 64) for i in range(9)] + [(576, 56)]
            for off, sz in zc:
                pltpu.sync_copy(zbuf.at[pl.ds(0, sz)],
                                acc.at[pl.ds(row0 + off, sz)])
            if with_cnt:
                pltpu.sync_copy(zcnt.at[pl.ds(0, ROWS_PT)],
                                cacc.at[pl.ds(row0, ROWS_PT)])
            plsc.subcore_barrier()

            @pl.loop(0, NCH, step=2)
            def _(k):
                h0 = load_idx(0, ebase0 + k * C)
                h1 = load_idx(1, ebase0 + (k + 1) * C)
                for h in h0:
                    h.wait()
                fixup(0, rel)
                g0 = pltpu.async_copy(y_hbm.at[giv[0]], buf[0], sem_g[0])
                for h in h1:
                    h.wait()
                fixup(1, rel)
                g0.wait()
                s0 = pltpu.async_copy(buf[0], acc.at[siv[0]], sem_s[0], add=True)
                if with_cnt:
                    c0 = pltpu.async_copy(onesv, cacc.at[siv[0]], sem_s[0],
                                          add=True)
                g1 = pltpu.async_copy(y_hbm.at[giv[1]], buf[1], sem_g[1])
                g1.wait()
                s1 = pltpu.async_copy(buf[1], acc.at[siv[1]], sem_s[1], add=True)
                if with_cnt:
                    c1 = pltpu.async_copy(onesv, cacc.at[siv[1]], sem_s[1],
                                          add=True)
                s0.wait()
                s1.wait()
                if with_cnt:
                    c0.wait()
                    c1.wait()

            plsc.subcore_barrier()

            fc = [(i * 80, 80) for i in range(7)] + [(560, 72)]
            for off, sz in fc:
                pltpu.sync_copy(acc.at[pl.ds(row0 + off, sz)],
                                buf[0].at[pl.ds(0, sz)])
                pltpu.sync_copy(buf[0].at[pl.ds(0, sz)],
                                a_hbm.at[rel, pl.ds(row0 + off, sz), :])
            if with_cnt:
                pltpu.sync_copy(cacc.at[pl.ds(row0, ROWS_PT)], cbuf)
                pltpu.sync_copy(cbuf,
                                cnt_hbm.at[pl.ds(rel * N_PAD + row0, ROWS_PT)])
            plsc.subcore_barrier()

    return body


def _sc_aggregate(edge_src, edge_dst, edge_type, y, with_cnt):
    mesh = plsc.VectorSubcoreMesh(core_axis_name="c", subcore_axis_name="s")
    out_type = jax.ShapeDtypeStruct((R, N_PAD, H), _f32)
    if with_cnt:
        out_type = (out_type, jax.ShapeDtypeStruct((R * N_PAD,), _f32))
    kern = pl.kernel(
        _make_agg_body(with_cnt),
        out_type=out_type,
        mesh=mesh,
        scratch_types=[
            [pltpu.VMEM((C,), _i32)] * 2,
            [pltpu.VMEM((C,), _i32)] * 2,
            [pltpu.VMEM((C,), _i32)] * 2,
            [pltpu.VMEM((C,), _i32)] * 2,
            [pltpu.VMEM((C,), _i32)] * 2,
            [pltpu.VMEM((C, H), _f32)] * 2,
            pltpu.VMEM((C,), _f32),
            pltpu.VMEM((64, H), _f32),
            pltpu.VMEM((ROWS_PT,), _f32),
            pltpu.VMEM((ROWS_PT,), _f32),
            [pltpu.SemaphoreType.DMA] * 2,
            [pltpu.SemaphoreType.DMA] * 2,
            [pltpu.SemaphoreType.DMA] * 2,
            pltpu.VMEM_SHARED((N_PAD, H), _f32),
            pltpu.VMEM_SHARED((N_PAD,), _f32),
        ],
    )
    return kern(edge_src, edge_dst, edge_type, y)


def _mm_body(x_ref, w_ref, b_ref, o_ref):
    j = pl.program_id(0)
    o = jnp.dot(x_ref[...], w_ref[0], preferred_element_type=_f32)

    @pl.when(j == R)
    def _():
        o_ref[0] = o + b_ref[...]

    @pl.when(j != R)
    def _():
        o_ref[0] = o


def _mm(x, ws, b):
    bn = 1000
    return pl.pallas_call(
        _mm_body,
        grid=(R + 1, N // bn),
        in_specs=[
            pl.BlockSpec((bn, F), lambda j, i: (i, 0)),
            pl.BlockSpec((1, F, H), lambda j, i: (j, 0, 0)),
            pl.BlockSpec((1, H), lambda j, i: (0, 0)),
        ],
        out_specs=pl.BlockSpec((1, bn, H), lambda j, i: (j, i, 0)),
        out_shape=jax.ShapeDtypeStruct((R + 1, N, H), _f32),
    )(x, ws, b.reshape(1, H))


def _comb_body(z_ref, a_ref, c_ref, o_ref):
    out = z_ref[0]
    for r in range(R):
        inv = 1.0 / jnp.maximum(c_ref[r], 1.0)
        out = out + a_ref[r] * inv
    o_ref[...] = jnp.maximum(out, 0.0)


def _combine(o_stacked, a, cnt):
    bn = 2000
    return pl.pallas_call(
        _comb_body,
        grid=(N // bn,),
        in_specs=[
            pl.BlockSpec((1, bn, H), lambda i: (R, i, 0)),
            pl.BlockSpec((R, bn, H), lambda i: (0, i, 0)),
            pl.BlockSpec((R, bn, 1), lambda i: (0, i, 0)),
        ],
        out_specs=pl.BlockSpec((bn, H), lambda i: (i, 0)),
        out_shape=jax.ShapeDtypeStruct((N, H), _f32),
    )(o_stacked, a, cnt)


def _pool_body(h_ref, b_ref, w_ref, bias_ref, o_ref, acc, cntg):
    i = pl.program_id(0)
    nb = pl.num_programs(0)

    @pl.when(i == 0)
    def _():
        acc[...] = jnp.zeros_like(acc)
        cntg[...] = jnp.zeros_like(cntg)

    ids = b_ref[0, 0, :]
    gid = lax.broadcasted_iota(_i32, (G, ids.shape[0]), 0)
    m = (ids[None, :] == gid).astype(_f32)
    acc[...] += jnp.dot(m, h_ref[...], preferred_element_type=_f32)
    cntg[...] += jnp.sum(m, axis=1, keepdims=True)

    @pl.when(i == nb - 1)
    def _():
        pooled = acc[...] / jnp.maximum(cntg[...], 1.0)
        logit = jnp.dot(pooled, w_ref[...], preferred_element_type=_f32)
        o_ref[...] = jax.nn.sigmoid(logit + bias_ref[0, 0])


def _pool_head(h, batch, lin_w, lin_b):
    bn = 1000
    batch3 = batch.reshape(N // bn, 1, bn)
    out = pl.pallas_call(
        _pool_body,
        grid=(N // bn,),
        in_specs=[
            pl.BlockSpec((bn, H), lambda i: (i, 0)),
            pl.BlockSpec((1, 1, bn), lambda i: (i, 0, 0)),
            pl.BlockSpec((H, 1), lambda i: (0, 0)),
            pl.BlockSpec((1, 1), lambda i: (0, 0)),
        ],
        out_specs=pl.BlockSpec((G, 1), lambda i: (0, 0)),
        out_shape=jax.ShapeDtypeStruct((G, 1), _f32),
        scratch_shapes=[
            pltpu.VMEM((G, H), _f32),
            pltpu.VMEM((G, 1), _f32),
        ],
    )(h, batch3, lin_w, lin_b.reshape(1, 1))
    return out.reshape(G)


def _layer(x, edge_src, edge_dst, edge_type, ws, b, cnt=None):
    o = _mm(x, ws, b)
    y = o[:R].reshape(R * N, H)
    if cnt is None:
        a, cnt_new = _sc_aggregate(edge_src, edge_dst, edge_type, y, True)
        cnt = cnt_new.reshape(R, N_PAD, 1)
    else:
        a = _sc_aggregate(edge_src, edge_dst, edge_type, y, False)
    return _combine(o, a, cnt), cnt


def kernel(x, edge_index, edge_type, batch, W1, root1, b1, W2, root2, b2,
           lin_w, lin_b):
    ws1 = jnp.concatenate([W1, root1[None]], axis=0)
    ws2 = jnp.concatenate([W2, root2[None]], axis=0)
    edge_src = edge_index[0]
    edge_dst = edge_index[1]
    h1, cnt = _layer(x, edge_src, edge_dst, edge_type, ws1, b1)
    h2, _ = _layer(h1, edge_src, edge_dst, edge_type, ws2, b2, cnt)
    return _pool_head(h2, batch, lin_w, lin_b)
